# Initial kernel scaffold; baseline (speedup 1.0000x reference)
import jax
import jax.numpy as jnp
from jax.experimental import pallas as pl


def kernel(x, params, edge_index):
    raise NotImplementedError("write your pallas kernel here")


# speedup vs baseline: 3.4574x; 3.4574x over previous
import functools

import jax
import jax.numpy as jnp
from jax import lax
from jax.experimental import pallas as pl
from jax.experimental.pallas import tpu as pltpu
from jax.experimental.pallas import tpu_sc as plsc

N = 10000
D = 128
E = 320000
OUT = 128

NC = 2
NS = 16
LANES = 16
NW = NC * NS
CH = 128
NCH = -(-E // (NW * CH))
E_PAD = NW * NCH * CH
STRIPE = 632
N_ACC = STRIPE * NS

@functools.cache
def _sc_mesh():
    return plsc.VectorSubcoreMesh(
        core_axis_name="c", subcore_axis_name="s",
        num_cores=NC, num_subcores=NS)


def _sc_agg_body(table, src_r, dst_r, zero_rows, out,
                 idx_s, idx_d, rows, accum, sem):
    c = lax.axis_index("c")
    s = lax.axis_index("s")
    w = s * NC + c
    pltpu.sync_copy(zero_rows.at[pl.ds(s * STRIPE, STRIPE)],
                    accum.at[pl.ds(s * STRIPE, STRIPE)])
    plsc.subcore_barrier()

    def chunk(k, carry):
        pltpu.sync_copy(src_r.at[w, k], idx_s)
        pltpu.sync_copy(dst_r.at[w, k], idx_d)
        pltpu.async_copy(table.at[idx_s], rows, sem).wait()
        pltpu.sync_copy(rows, accum.at[idx_d], add=True)
        return carry

    lax.fori_loop(0, NCH, chunk, 0)
    plsc.subcore_barrier()
    pltpu.sync_copy(accum.at[pl.ds(s * STRIPE, STRIPE)],
                    out.at[c, pl.ds(s * STRIPE, STRIPE)])


def _sc_deg_body(dst_r, zero_rows, ones_rows, out, idx_d, ones_t, accum):
    c = lax.axis_index("c")
    s = lax.axis_index("s")
    w = s * NC + c
    pltpu.sync_copy(zero_rows.at[pl.ds(s * STRIPE, STRIPE)],
                    accum.at[pl.ds(s * STRIPE, STRIPE)])
    pltpu.sync_copy(ones_rows, ones_t)
    plsc.subcore_barrier()

    def chunk(k, carry):
        pltpu.sync_copy(dst_r.at[w, k], idx_d)
        pltpu.sync_copy(ones_t, accum.at[idx_d], add=True)
        return carry

    lax.fori_loop(0, NCH, chunk, 0)
    plsc.subcore_barrier()
    pltpu.sync_copy(accum.at[pl.ds(s * STRIPE, STRIPE)],
                    out.at[c, pl.ds(s * STRIPE, STRIPE)])


@functools.cache
def _sc_kernels():
    mesh = _sc_mesh()
    agg = pl.kernel(
        _sc_agg_body,
        out_type=jax.ShapeDtypeStruct((NC, N_ACC, D), jnp.float32),
        mesh=mesh,
        scratch_types=[
            pltpu.VMEM((CH,), jnp.int32),
            pltpu.VMEM((CH,), jnp.int32),
            pltpu.VMEM((CH, D), jnp.float32),
            pltpu.VMEM_SHARED((N_ACC, D), jnp.float32),
            pltpu.SemaphoreType.DMA,
        ],
    )
    deg = pl.kernel(
        _sc_deg_body,
        out_type=jax.ShapeDtypeStruct((NC, N_ACC, D), jnp.float32),
        mesh=mesh,
        scratch_types=[
            pltpu.VMEM((CH,), jnp.int32),
            pltpu.VMEM((CH, D), jnp.float32),
            pltpu.VMEM_SHARED((N_ACC, D), jnp.float32),
        ],
    )
    return deg, agg


TB = 1000


def _ln_relu(z):
    z = jnp.maximum(z, 0.0)
    mu = jnp.mean(z, axis=-1, keepdims=True)
    cen = z - mu
    var = jnp.mean(cen * cen, axis=-1, keepdims=True)
    return cen * lax.rsqrt(var + 1e-5)


def _tc_in_body(x_ref, w_ref, b_ref, o_ref):
    o_ref[...] = (jnp.dot(x_ref[...], w_ref[...],
                          preferred_element_type=jnp.float32) + b_ref[...])


def _tc_in(x, w, b):
    return pl.pallas_call(
        _tc_in_body,
        grid=(N // TB,),
        in_specs=[pl.BlockSpec((TB, D), lambda i: (i, 0)),
                  pl.BlockSpec((D, D), lambda i: (0, 0)),
                  pl.BlockSpec((1, D), lambda i: (0, 0))],
        out_specs=pl.BlockSpec((TB, D), lambda i: (i, 0)),
        out_shape=jax.ShapeDtypeStruct((N, D), jnp.float32),
    )(x, w, b)


def _layer_block(a0, a1, d0, d1, h, wn, b, ws):
    agg = a0 + a1
    deg = d0 + d1
    mean = agg / jnp.maximum(deg, 1.0)
    z = (jnp.dot(mean, wn, preferred_element_type=jnp.float32)
         + jnp.dot(h, ws, preferred_element_type=jnp.float32) + b)
    return _ln_relu(z)


def _tc_layer_body(a0_ref, a1_ref, d0_ref, d1_ref, h_ref, wn_ref, b_ref,
                   ws_ref, o_ref):
    o_ref[...] = _layer_block(a0_ref[0], a1_ref[0], d0_ref[0], d1_ref[0],
                              h_ref[...], wn_ref[...], b_ref[...], ws_ref[...])


def _agg_specs():
    return [pl.BlockSpec((1, TB, D), lambda i: (0, i, 0)),
            pl.BlockSpec((1, TB, D), lambda i: (1, i, 0)),
            pl.BlockSpec((1, TB, D), lambda i: (0, i, 0)),
            pl.BlockSpec((1, TB, D), lambda i: (1, i, 0))]


def _wspec(shape):
    return pl.BlockSpec(shape, lambda i: tuple(0 for _ in shape))


def _tc_layer(agg, deg, h, wn, b, ws):
    return pl.pallas_call(
        _tc_layer_body,
        grid=(N // TB,),
        in_specs=_agg_specs() + [
            pl.BlockSpec((TB, D), lambda i: (i, 0)),
            _wspec((D, D)), _wspec((1, D)), _wspec((D, D)),
        ],
        out_specs=pl.BlockSpec((TB, D), lambda i: (i, 0)),
        out_shape=jax.ShapeDtypeStruct((N, D), jnp.float32),
    )(agg, agg, deg, deg, h, wn, b, ws)


def _tc_final_body(a0_ref, a1_ref, d0_ref, d1_ref, h_ref, wn_ref, b_ref,
                   ws_ref, wo1_ref, bo1_ref, wo2_ref, bo2_ref, wp1_ref,
                   bp1_ref, wp2_ref, bp2_ref, wp3_ref, bp3_ref,
                   emb_ref, perf_ref):
    h3 = _layer_block(a0_ref[0], a1_ref[0], d0_ref[0], d1_ref[0], h_ref[...],
                      wn_ref[...], b_ref[...], ws_ref[...])
    t = jnp.maximum(jnp.dot(h3, wo1_ref[...],
                            preferred_element_type=jnp.float32)
                    + bo1_ref[...], 0.0)
    emb = jnp.dot(t, wo2_ref[...], preferred_element_type=jnp.float32) \
        + bo2_ref[...]
    emb_ref[...] = emb
    u = jnp.maximum(jnp.dot(emb, wp1_ref[...],
                            preferred_element_type=jnp.float32)
                    + bp1_ref[...], 0.0)
    v = jnp.maximum(jnp.dot(u, wp2_ref[...],
                            preferred_element_type=jnp.float32)
                    + bp2_ref[...], 0.0)
    perf_ref[...] = jnp.dot(v, wp3_ref[...],
                            preferred_element_type=jnp.float32) + bp3_ref[...]


def _tc_final(agg, deg, h, wn, b, ws, wo1, bo1, wo2, bo2, wp1a, bp1, wp2,
              bp2, wp3, bp3):
    return pl.pallas_call(
        _tc_final_body,
        grid=(N // TB,),
        in_specs=_agg_specs() + [
            pl.BlockSpec((TB, D), lambda i: (i, 0)),
            _wspec((D, D)), _wspec((1, D)), _wspec((D, D)),
            _wspec((D, D // 2)), _wspec((1, D // 2)),
            _wspec((D // 2, OUT)), _wspec((1, OUT)),
            _wspec((OUT, 128)), _wspec((1, 128)),
            _wspec((128, 64)), _wspec((1, 64)),
            _wspec((64, 1)), _wspec((1, 1)),
        ],
        out_specs=[pl.BlockSpec((TB, OUT), lambda i: (i, 0)),
                   pl.BlockSpec((TB, 1), lambda i: (i, 0))],
        out_shape=[jax.ShapeDtypeStruct((N, OUT), jnp.float32),
                   jax.ShapeDtypeStruct((N, 1), jnp.float32)],
    )(agg, agg, deg, deg, h, wn, b, ws, wo1, bo1, wo2, bo2, wp1a, bp1,
      wp2, bp2, wp3, bp3)


def kernel(x, params, edge_index):
    p = params
    src = edge_index[0]
    dst = edge_index[1]
    pad = E_PAD - E
    src_r = jnp.concatenate(
        [src, jnp.zeros((pad,), jnp.int32)]).reshape(NW, NCH, CH)
    dst_r = jnp.concatenate(
        [dst, jnp.full((pad,), N, jnp.int32)]).reshape(NW, NCH, CH)
    zero_rows = jnp.zeros((N_ACC, D), jnp.float32)
    ones_rows = jnp.ones((CH, D), jnp.float32)

    h = _tc_in(x, p['W_in'], p['b_in'].reshape(1, D))

    _sc_deg, _sc_agg = _sc_kernels()
    lp = p['layers']
    deg = _sc_deg(dst_r, zero_rows, ones_rows)
    agg = _sc_agg(h, src_r, dst_r, zero_rows)
    h = _tc_layer(agg, deg, h, lp[0]['W_neigh'], lp[0]['b'].reshape(1, D),
                  lp[0]['W_self'])
    agg = _sc_agg(h, src_r, dst_r, zero_rows)
    h = _tc_layer(agg, deg, h, lp[1]['W_neigh'], lp[1]['b'].reshape(1, D),
                  lp[1]['W_self'])
    agg = _sc_agg(h, src_r, dst_r, zero_rows)
    emb, perf = _tc_final(
        agg, deg, h, lp[2]['W_neigh'], lp[2]['b'].reshape(1, D),
        lp[2]['W_self'],
        p['W_o1'], p['b_o1'].reshape(1, D // 2),
        p['W_o2'], p['b_o2'].reshape(1, OUT),
        p['W_p1'][:OUT], p['b_p1'].reshape(1, 128),
        p['W_p2'], p['b_p2'].reshape(1, 64),
        p['W_p3'], p['b_p3'].reshape(1, 1))
    ctx = jnp.zeros((N, 64), jnp.float32)
    return emb, ctx, perf


# speedup vs baseline: 9.8393x; 2.8458x over previous
import functools

import jax
import jax.numpy as jnp
from jax import lax
from jax.experimental import pallas as pl
from jax.experimental.pallas import tpu as pltpu
from jax.experimental.pallas import tpu_sc as plsc

N = 10000
D = 128
E = 320000
OUT = 128

NC = 2
NS = 16
LANES = 16
NW = NC * NS
CH = 128
NCH = 4 * (-(-E // (NW * CH * 4)))
HALF = NCH // 2
PAIR_PH = HALF // 2
E_PAD = NW * NCH * CH
STRIPE = 632
N_ACC = STRIPE * NS

@functools.cache
def _sc_mesh():
    return plsc.VectorSubcoreMesh(
        core_axis_name="c", subcore_axis_name="s",
        num_cores=NC, num_subcores=NS)


def _sc_agg_body(table, src_r, dst_r, zero_rows, out,
                 idx_s, idx_d, rows0, rows1, accum, sem0, sem1):
    c = lax.axis_index("c")
    s = lax.axis_index("s")
    w = s * NC + c
    pltpu.sync_copy(zero_rows.at[pl.ds(s * STRIPE, STRIPE)],
                    accum.at[pl.ds(s * STRIPE, STRIPE)])
    plsc.subcore_barrier()

    for half in range(2):
        base = half * HALF
        pltpu.sync_copy(src_r.at[w, pl.ds(base, HALF)], idx_s)
        pltpu.sync_copy(dst_r.at[w, pl.ds(base, HALF)], idx_d)
        pltpu.async_copy(table.at[idx_s.at[0]], rows0, sem0)

        def pair(i, carry):
            k0 = 2 * i
            k1 = k0 + 1
            pltpu.async_copy(table.at[idx_s.at[k1]], rows1, sem1)
            pltpu.make_async_copy(table.at[idx_s.at[k0]], rows0, sem0).wait()
            pltpu.sync_copy(rows0, accum.at[idx_d.at[k0]], add=True)

            @pl.when(i < PAIR_PH - 1)
            def _():
                pltpu.async_copy(table.at[idx_s.at[k0 + 2]], rows0, sem0)

            pltpu.make_async_copy(table.at[idx_s.at[k1]], rows1, sem1).wait()
            pltpu.sync_copy(rows1, accum.at[idx_d.at[k1]], add=True)
            return carry

        lax.fori_loop(0, PAIR_PH, pair, 0)
    plsc.subcore_barrier()
    pltpu.sync_copy(accum.at[pl.ds(s * STRIPE, STRIPE)],
                    out.at[c, pl.ds(s * STRIPE, STRIPE)])


DEG_G = 8


def _sc_deg_body(dst_r, zero_rows, ones_rows, out, idx_d, ones_t, accum, sem):
    c = lax.axis_index("c")
    s = lax.axis_index("s")
    w = s * NC + c
    pltpu.sync_copy(zero_rows.at[pl.ds(s * STRIPE, STRIPE)],
                    accum.at[pl.ds(s * STRIPE, STRIPE)])
    pltpu.sync_copy(ones_rows, ones_t)
    pltpu.sync_copy(dst_r.at[w], idx_d)
    plsc.subcore_barrier()

    def group(g, carry):
        base = g * DEG_G
        for j in range(DEG_G):
            pltpu.async_copy(ones_t, accum.at[idx_d.at[base + j]], sem,
                             add=True)
        for j in range(DEG_G):
            pltpu.make_async_copy(ones_t, accum.at[idx_d.at[base + j]],
                                  sem).wait()
        return carry

    lax.fori_loop(0, NCH // DEG_G, group, 0)
    plsc.subcore_barrier()
    pltpu.sync_copy(accum.at[pl.ds(s * STRIPE, STRIPE)],
                    out.at[c, pl.ds(s * STRIPE, STRIPE)])


@functools.cache
def _sc_kernels():
    mesh = _sc_mesh()
    agg = pl.kernel(
        _sc_agg_body,
        out_type=jax.ShapeDtypeStruct((NC, N_ACC, D), jnp.float32),
        mesh=mesh,
        scratch_types=[
            pltpu.VMEM((HALF, CH), jnp.int32),
            pltpu.VMEM((HALF, CH), jnp.int32),
            pltpu.VMEM((CH, D), jnp.float32),
            pltpu.VMEM((CH, D), jnp.float32),
            pltpu.VMEM_SHARED((N_ACC, D), jnp.float32),
            pltpu.SemaphoreType.DMA,
            pltpu.SemaphoreType.DMA,
        ],
    )
    deg = pl.kernel(
        _sc_deg_body,
        out_type=jax.ShapeDtypeStruct((NC, N_ACC, D), jnp.float32),
        mesh=mesh,
        scratch_types=[
            pltpu.VMEM((NCH, CH), jnp.int32),
            pltpu.VMEM((CH, D), jnp.float32),
            pltpu.VMEM_SHARED((N_ACC, D), jnp.float32),
            pltpu.SemaphoreType.DMA,
        ],
    )
    return deg, agg


TB = 1000


def _ln_relu(z):
    z = jnp.maximum(z, 0.0)
    mu = jnp.mean(z, axis=-1, keepdims=True)
    cen = z - mu
    var = jnp.mean(cen * cen, axis=-1, keepdims=True)
    return cen * lax.rsqrt(var + 1e-5)


def _tc_in_body(x_ref, w_ref, b_ref, o_ref):
    o_ref[...] = (jnp.dot(x_ref[...], w_ref[...],
                          preferred_element_type=jnp.float32) + b_ref[...])


def _tc_in(x, w, b):
    return pl.pallas_call(
        _tc_in_body,
        grid=(N // TB,),
        in_specs=[pl.BlockSpec((TB, D), lambda i: (i, 0)),
                  pl.BlockSpec((D, D), lambda i: (0, 0)),
                  pl.BlockSpec((1, D), lambda i: (0, 0))],
        out_specs=pl.BlockSpec((TB, D), lambda i: (i, 0)),
        out_shape=jax.ShapeDtypeStruct((N, D), jnp.float32),
    )(x, w, b)


def _layer_block(a0, a1, d0, d1, h, wn, b, ws):
    agg = a0 + a1
    deg = d0 + d1
    mean = agg / jnp.maximum(deg, 1.0)
    z = (jnp.dot(mean, wn, preferred_element_type=jnp.float32)
         + jnp.dot(h, ws, preferred_element_type=jnp.float32) + b)
    return _ln_relu(z)


def _tc_layer_body(a0_ref, a1_ref, d0_ref, d1_ref, h_ref, wn_ref, b_ref,
                   ws_ref, o_ref):
    o_ref[...] = _layer_block(a0_ref[0], a1_ref[0], d0_ref[0], d1_ref[0],
                              h_ref[...], wn_ref[...], b_ref[...], ws_ref[...])


def _agg_specs():
    return [pl.BlockSpec((1, TB, D), lambda i: (0, i, 0)),
            pl.BlockSpec((1, TB, D), lambda i: (1, i, 0)),
            pl.BlockSpec((1, TB, D), lambda i: (0, i, 0)),
            pl.BlockSpec((1, TB, D), lambda i: (1, i, 0))]


def _wspec(shape):
    return pl.BlockSpec(shape, lambda i: tuple(0 for _ in shape))


def _tc_layer(agg, deg, h, wn, b, ws):
    return pl.pallas_call(
        _tc_layer_body,
        grid=(N // TB,),
        in_specs=_agg_specs() + [
            pl.BlockSpec((TB, D), lambda i: (i, 0)),
            _wspec((D, D)), _wspec((1, D)), _wspec((D, D)),
        ],
        out_specs=pl.BlockSpec((TB, D), lambda i: (i, 0)),
        out_shape=jax.ShapeDtypeStruct((N, D), jnp.float32),
    )(agg, agg, deg, deg, h, wn, b, ws)


def _tc_final_body(a0_ref, a1_ref, d0_ref, d1_ref, h_ref, wn_ref, b_ref,
                   ws_ref, wo1_ref, bo1_ref, wo2_ref, bo2_ref, wp1_ref,
                   bp1_ref, wp2_ref, bp2_ref, wp3_ref, bp3_ref,
                   emb_ref, perf_ref):
    h3 = _layer_block(a0_ref[0], a1_ref[0], d0_ref[0], d1_ref[0], h_ref[...],
                      wn_ref[...], b_ref[...], ws_ref[...])
    t = jnp.maximum(jnp.dot(h3, wo1_ref[...],
                            preferred_element_type=jnp.float32)
                    + bo1_ref[...], 0.0)
    emb = jnp.dot(t, wo2_ref[...], preferred_element_type=jnp.float32) \
        + bo2_ref[...]
    emb_ref[...] = emb
    u = jnp.maximum(jnp.dot(emb, wp1_ref[...],
                            preferred_element_type=jnp.float32)
                    + bp1_ref[...], 0.0)
    v = jnp.maximum(jnp.dot(u, wp2_ref[...],
                            preferred_element_type=jnp.float32)
                    + bp2_ref[...], 0.0)
    perf_ref[...] = jnp.dot(v, wp3_ref[...],
                            preferred_element_type=jnp.float32) + bp3_ref[...]


def _tc_final(agg, deg, h, wn, b, ws, wo1, bo1, wo2, bo2, wp1a, bp1, wp2,
              bp2, wp3, bp3):
    return pl.pallas_call(
        _tc_final_body,
        grid=(N // TB,),
        in_specs=_agg_specs() + [
            pl.BlockSpec((TB, D), lambda i: (i, 0)),
            _wspec((D, D)), _wspec((1, D)), _wspec((D, D)),
            _wspec((D, D // 2)), _wspec((1, D // 2)),
            _wspec((D // 2, OUT)), _wspec((1, OUT)),
            _wspec((OUT, 128)), _wspec((1, 128)),
            _wspec((128, 64)), _wspec((1, 64)),
            _wspec((64, 1)), _wspec((1, 1)),
        ],
        out_specs=[pl.BlockSpec((TB, OUT), lambda i: (i, 0)),
                   pl.BlockSpec((TB, 1), lambda i: (i, 0))],
        out_shape=[jax.ShapeDtypeStruct((N, OUT), jnp.float32),
                   jax.ShapeDtypeStruct((N, 1), jnp.float32)],
    )(agg, agg, deg, deg, h, wn, b, ws, wo1, bo1, wo2, bo2, wp1a, bp1,
      wp2, bp2, wp3, bp3)


def kernel(x, params, edge_index):
    p = params
    src = edge_index[0]
    dst = edge_index[1]
    pad = E_PAD - E
    spare = N_ACC - N
    src_r = jnp.concatenate(
        [src, (jnp.arange(pad, dtype=jnp.int32) * 37) % N]).reshape(
            NW, NCH, CH)
    dst_r = jnp.concatenate(
        [dst, N + (jnp.arange(pad, dtype=jnp.int32) % spare)]).reshape(
            NW, NCH, CH)
    zero_rows = jnp.zeros((N_ACC, D), jnp.float32)
    ones_rows = jnp.ones((CH, D), jnp.float32)

    h = _tc_in(x, p['W_in'], p['b_in'].reshape(1, D))

    _sc_deg, _sc_agg = _sc_kernels()
    lp = p['layers']
    deg = _sc_deg(dst_r, zero_rows, ones_rows)
    agg = _sc_agg(h, src_r, dst_r, zero_rows)
    h = _tc_layer(agg, deg, h, lp[0]['W_neigh'], lp[0]['b'].reshape(1, D),
                  lp[0]['W_self'])
    agg = _sc_agg(h, src_r, dst_r, zero_rows)
    h = _tc_layer(agg, deg, h, lp[1]['W_neigh'], lp[1]['b'].reshape(1, D),
                  lp[1]['W_self'])
    agg = _sc_agg(h, src_r, dst_r, zero_rows)
    emb, perf = _tc_final(
        agg, deg, h, lp[2]['W_neigh'], lp[2]['b'].reshape(1, D),
        lp[2]['W_self'],
        p['W_o1'], p['b_o1'].reshape(1, D // 2),
        p['W_o2'], p['b_o2'].reshape(1, OUT),
        p['W_p1'][:OUT], p['b_p1'].reshape(1, 128),
        p['W_p2'], p['b_p2'].reshape(1, 64),
        p['W_p3'], p['b_p3'].reshape(1, 1))
    ctx = jnp.zeros((N, 64), jnp.float32)
    return emb, ctx, perf
